# Initial kernel scaffold; baseline (speedup 1.0000x reference)
import jax
import jax.numpy as jnp
from jax.experimental import pallas as pl


def kernel(weeks, seasonal_bias):
    raise NotImplementedError("write your pallas kernel here")


# speedup vs baseline: 3.0746x; 3.0746x over previous
import jax
import jax.numpy as jnp
from jax import lax
from jax.experimental import pallas as pl
from jax.experimental.pallas import tpu as pltpu
from jax.experimental.pallas import tpu_sc as plsc

_ROW = 7 * 24
_NW = 32
_CHUNK = 128


def kernel(weeks, seasonal_bias):
    n = weeks.shape[0]
    table = seasonal_bias.reshape(seasonal_bias.shape[0], _ROW)
    per_w = n // _NW
    chunks = per_w // _CHUNK
    idx = weeks.astype(jnp.int32)

    mesh = plsc.VectorSubcoreMesh(core_axis_name="c", subcore_axis_name="s")

    @pl.kernel(
        out_type=jax.ShapeDtypeStruct((n, _ROW), table.dtype),
        mesh=mesh,
        scratch_types=[
            pltpu.VMEM((per_w,), jnp.int32),
            pltpu.VMEM((_CHUNK, _ROW), jnp.float32),
            pltpu.SemaphoreType.DMA,
        ],
        compiler_params=pltpu.CompilerParams(use_tc_tiling_on_sc=False),
    )
    def gather_kernel(table_hbm, idx_hbm, out_hbm, idx_v, rows_v, sem):
        wid = lax.axis_index("s") * 2 + lax.axis_index("c")
        pltpu.sync_copy(idx_hbm.at[pl.ds(wid * per_w, per_w)], idx_v)

        @pl.loop(0, chunks)
        def _(k):
            pltpu.async_copy(
                table_hbm.at[idx_v.at[pl.ds(k * _CHUNK, _CHUNK)]], rows_v, sem
            ).wait()
            pltpu.sync_copy(
                rows_v, out_hbm.at[pl.ds(wid * per_w + k * _CHUNK, _CHUNK)]
            )

    out = gather_kernel(table, idx)
    return out.reshape(n, 7, 24)


# speedup vs baseline: 4.0860x; 1.3289x over previous
import jax
import jax.numpy as jnp
from jax import lax
from jax.experimental import pallas as pl
from jax.experimental.pallas import tpu as pltpu
from jax.experimental.pallas import tpu_sc as plsc

_ROW = 7 * 24
_NW = 32
_CHUNK = 128


def kernel(weeks, seasonal_bias):
    n = weeks.shape[0]
    v = seasonal_bias.shape[0]
    table = seasonal_bias.reshape(v, _ROW)
    per_w = n // _NW
    chunks = per_w // _CHUNK
    idx = weeks.astype(jnp.int32)

    mesh = plsc.VectorSubcoreMesh(core_axis_name="c", subcore_axis_name="s")

    @pl.kernel(
        out_type=jax.ShapeDtypeStruct((n, _ROW), table.dtype),
        mesh=mesh,
        scratch_types=[
            pltpu.VMEM_SHARED((v, _ROW), jnp.float32),
            pltpu.VMEM((per_w,), jnp.int32),
            pltpu.VMEM((per_w, _ROW), jnp.float32),
            pltpu.SemaphoreType.DMA((chunks,)),
            pltpu.SemaphoreType.DMA((chunks,)),
            pltpu.SemaphoreType.DMA,
        ],
        compiler_params=pltpu.CompilerParams(use_tc_tiling_on_sc=False),
    )
    def gather_kernel(
        table_hbm, idx_hbm, out_hbm, table_v, idx_v, rows_v, gsem, osem, tsem
    ):
        wid = lax.axis_index("s") * 2 + lax.axis_index("c")
        base = wid * per_w
        tcopy = pltpu.make_async_copy(table_hbm, table_v, tsem)
        tcopy.start()
        pltpu.sync_copy(idx_hbm.at[pl.ds(base, per_w)], idx_v)
        tcopy.wait()

        gathers = []
        for k in range(chunks):
            g = pltpu.make_async_copy(
                table_v.at[idx_v.at[pl.ds(k * _CHUNK, _CHUNK)]],
                rows_v.at[pl.ds(k * _CHUNK, _CHUNK)],
                gsem.at[k],
            )
            g.start()
            gathers.append(g)
        outs = []
        for k in range(chunks):
            gathers[k].wait()
            o = pltpu.make_async_copy(
                rows_v.at[pl.ds(k * _CHUNK, _CHUNK)],
                out_hbm.at[pl.ds(base + k * _CHUNK, _CHUNK)],
                osem.at[k],
            )
            o.start()
            outs.append(o)
        for o in outs:
            o.wait()

    out = gather_kernel(table, idx)
    return out.reshape(n, 7, 24)
